# baseline (speedup 1.0000x reference)
import jax
import jax.numpy as jnp
from jax import lax
from jax.experimental import pallas as pl
from jax.experimental.pallas import tpu as pltpu
from jax.experimental.pallas import tpu_sc as plsc

BATCH = 4
MAX_LEN = 2048
EMBED = 1024
NC, NS, L = 2, 16, 16
NW = NC * NS
T_PER_W = MAX_LEN // NW
CT = 32
NCHUNK = T_PER_W // CT
VREGS_PER_ROW = EMBED // L


def _body(seq_hbm, tok_hbm, pos_hbm, out_hbm, idx_v, rows_v, pos_v, sem):
    wid = lax.axis_index("s") * NC + lax.axis_index("c")
    tw0 = wid * T_PER_W
    for c in range(NCHUNK):
        t0 = tw0 + c * CT
        pltpu.sync_copy(pos_hbm.at[pl.ds(t0, CT)], pos_v)
        for b in range(BATCH):
            pltpu.sync_copy(seq_hbm.at[b, pl.ds(t0, CT)], idx_v)
            pltpu.async_copy(tok_hbm.at[idx_v], rows_v, sem).wait()

            def add_row(r, carry):
                for j in range(VREGS_PER_ROW):
                    sl = pl.ds(j * L, L)
                    rows_v[r, sl] = rows_v[r, sl] + pos_v[r, sl]
                return carry

            lax.fori_loop(0, CT, add_row, 0)
            pltpu.sync_copy(rows_v, out_hbm.at[b, pl.ds(t0, CT)])


def kernel(sequence, token_weight, position_weight):
    mesh = plsc.VectorSubcoreMesh(core_axis_name="c", subcore_axis_name="s")
    f = pl.kernel(
        _body,
        out_type=jax.ShapeDtypeStruct((BATCH, MAX_LEN, EMBED), jnp.float32),
        mesh=mesh,
        scratch_types=[
            pltpu.VMEM((CT,), jnp.int32),
            pltpu.VMEM((CT, EMBED), jnp.float32),
            pltpu.VMEM((CT, EMBED), jnp.float32),
            pltpu.SemaphoreType.DMA,
        ],
    )
    return f(sequence, token_weight, position_weight)


# speedup vs baseline: 1.5132x; 1.5132x over previous
import jax
import jax.numpy as jnp
from jax import lax
from jax.experimental import pallas as pl
from jax.experimental.pallas import tpu as pltpu
from jax.experimental.pallas import tpu_sc as plsc

BATCH = 4
MAX_LEN = 2048
EMBED = 1024
NC, NS, L = 2, 16, 16
NW = NC * NS
T_PER_W = MAX_LEN // NW
CT = 8
NCHUNK = T_PER_W // CT
VREGS_PER_ROW = EMBED // L


def _body(seq_hbm, tok_hbm, pos_hbm, out_hbm,
          idx_v, rows0, rows1, pos0, pos1, gsem0, gsem1, ssem0, ssem1):
    wid = lax.axis_index("s") * NC + lax.axis_index("c")
    tw0 = wid * T_PER_W
    for b in range(BATCH):
        pltpu.sync_copy(seq_hbm.at[b, pl.ds(tw0, T_PER_W)], idx_v.at[b])

    rows = [rows0, rows1]
    pos = [pos0, pos1]
    gsem = [gsem0, gsem1]
    ssem = [ssem0, ssem1]

    def start_unit(c):
        buf = c % 2
        t0 = tw0 + c * CT
        descs = [pltpu.async_copy(pos_hbm.at[pl.ds(t0, CT)], pos[buf], gsem[buf])]
        for b in range(BATCH):
            descs.append(pltpu.async_copy(
                tok_hbm.at[idx_v.at[b, pl.ds(c * CT, CT)]],
                rows[buf].at[b], gsem[buf]))
        return descs

    pend_g = {0: start_unit(0)}
    pend_s = {}
    for c in range(NCHUNK):
        buf = c % 2
        nxt = c + 1
        if nxt < NCHUNK:
            for d in pend_s.pop(nxt % 2, ()):
                d.wait()
            pend_g[nxt] = start_unit(nxt)
        for d in pend_g.pop(c):
            d.wait()

        def add_j(j, carry, _buf=buf):
            sl = pl.ds(j * L, L)
            for r in range(CT):
                p = pos[_buf][r, sl]
                for b in range(BATCH):
                    rows[_buf][b, r, sl] = rows[_buf][b, r, sl] + p
            return carry

        lax.fori_loop(0, VREGS_PER_ROW, add_j, 0)

        t0 = tw0 + c * CT
        pend_s[buf] = [
            pltpu.async_copy(rows[buf].at[b], out_hbm.at[b, pl.ds(t0, CT)],
                             ssem[buf])
            for b in range(BATCH)
        ]
    for descs in pend_s.values():
        for d in descs:
            d.wait()


def kernel(sequence, token_weight, position_weight):
    mesh = plsc.VectorSubcoreMesh(core_axis_name="c", subcore_axis_name="s")
    f = pl.kernel(
        _body,
        out_type=jax.ShapeDtypeStruct((BATCH, MAX_LEN, EMBED), jnp.float32),
        mesh=mesh,
        scratch_types=[
            pltpu.VMEM((BATCH, T_PER_W), jnp.int32),
            pltpu.VMEM((BATCH, CT, EMBED), jnp.float32),
            pltpu.VMEM((BATCH, CT, EMBED), jnp.float32),
            pltpu.VMEM((CT, EMBED), jnp.float32),
            pltpu.VMEM((CT, EMBED), jnp.float32),
            pltpu.SemaphoreType.DMA,
            pltpu.SemaphoreType.DMA,
            pltpu.SemaphoreType.DMA,
            pltpu.SemaphoreType.DMA,
        ],
    )
    return f(sequence, token_weight, position_weight)
